# Initial kernel scaffold; baseline (speedup 1.0000x reference)
import jax
import jax.numpy as jnp
from jax.experimental import pallas as pl
from jax.experimental.pallas import tpu as pltpu

_E = 8
_C = 128
_NY = 64
_NX = 128
_P = 2
_N = _NY * _NX
_T = 512


def _fused(x_ref, eta_ref, W_ref, b_ref, Wt_ref, Tmat_ref, out_ref, eta_out_ref):
    x_t = x_ref[...]
    eta_t = eta_ref[0, :]

    logits = jax.lax.dot_general(
        x_t, Wt_ref[...], (((0,), (0,)), ((), ())),
        preferred_element_type=jnp.float32)
    eidx = jax.lax.broadcasted_iota(jnp.int32, (_T, _E), 1)
    oh_old = (eta_t[:, None] == eidx).astype(jnp.float32)
    logits = logits + jax.lax.dot_general(
        oh_old, Tmat_ref[...], (((1,), (0,)), ((), ())),
        preferred_element_type=jnp.float32)
    new_eta = jnp.argmax(logits, axis=1).astype(jnp.int32)
    eta_out_ref[0, :] = new_eta

    mask = (new_eta[None, :] == jax.lax.broadcasted_iota(
        jnp.int32, (_E, _T), 0)).astype(jnp.float32)
    badd = jax.lax.dot_general(
        b_ref[...], mask, (((1,), (0,)), ((), ())),
        preferred_element_type=jnp.float32)

    for p in range(_P):
        acc = badd[p]
        for e in range(_E):
            y = jax.lax.dot_general(
                W_ref[p, e], x_t, (((1,), (0,)), ((), ())),
                preferred_element_type=jnp.float32)
            acc = acc + y * mask[e][None, :]
        out_ref[p] = acc


def kernel(x, eta, W, b, Wt, Tmat):
    x2 = x.reshape(_C, _N)
    eta2 = eta.reshape(1, _N).astype(jnp.int32)
    grid = (_N // _T,)
    out, new_eta = pl.pallas_call(
        _fused,
        grid=grid,
        in_specs=[
            pl.BlockSpec((_C, _T), lambda i: (0, i)),
            pl.BlockSpec((1, _T), lambda i: (0, i)),
            pl.BlockSpec((_P, _E, _C, _C), lambda i: (0, 0, 0, 0)),
            pl.BlockSpec((_P, _E, _C), lambda i: (0, 0, 0)),
            pl.BlockSpec((_C, _E), lambda i: (0, 0)),
            pl.BlockSpec((_E, _E), lambda i: (0, 0)),
        ],
        out_specs=[
            pl.BlockSpec((_P, _C, _T), lambda i: (0, 0, i)),
            pl.BlockSpec((1, _T), lambda i: (0, i)),
        ],
        out_shape=[
            jax.ShapeDtypeStruct((_P, _C, _N), jnp.float32),
            jax.ShapeDtypeStruct((1, _N), jnp.int32),
        ],
        compiler_params=pltpu.CompilerParams(
            dimension_semantics=("arbitrary",)),
    )(x2, eta2, W, b, Wt, Tmat)
    return out.reshape(_P, _C, _NY, _NX), new_eta.reshape(_NY, _NX)


# speedup vs baseline: 1.0977x; 1.0977x over previous
import jax
import jax.numpy as jnp
from jax.experimental import pallas as pl
from jax.experimental.pallas import tpu as pltpu

_E = 8
_C = 128
_NY = 64
_NX = 128
_P = 2
_N = _NY * _NX
_T = 512


def _fused(x_ref, eta_ref, W_ref, b_ref, Wt_ref, Tmat_ref, out_ref, eta_out_ref):
    x_t = x_ref[...]
    eta_t = eta_ref[0, :]

    logits = jax.lax.dot_general(
        x_t, Wt_ref[...], (((0,), (0,)), ((), ())),
        preferred_element_type=jnp.float32)
    tadd = jnp.zeros((_T, _E), jnp.float32)
    for k in range(_E):
        tadd = jnp.where(eta_t[:, None] == k, Tmat_ref[k][None, :], tadd)
    logits = logits + tadd
    new_eta = jnp.argmax(logits, axis=1).astype(jnp.int32)
    eta_out_ref[0, :] = new_eta

    mask = (new_eta[None, :] == jax.lax.broadcasted_iota(
        jnp.int32, (_E, _T), 0)).astype(jnp.float32)
    badd = jax.lax.dot_general(
        b_ref[...], mask, (((1,), (0,)), ((), ())),
        preferred_element_type=jnp.float32)

    for p in range(_P):
        acc = badd[p]
        for e in range(_E):
            y = jax.lax.dot_general(
                W_ref[p, e], x_t, (((1,), (0,)), ((), ())),
                preferred_element_type=jnp.float32)
            acc = acc + y * mask[e][None, :]
        out_ref[p] = acc


def kernel(x, eta, W, b, Wt, Tmat):
    x2 = x.reshape(_C, _N)
    eta2 = eta.reshape(1, _N).astype(jnp.int32)
    grid = (_N // _T,)
    out, new_eta = pl.pallas_call(
        _fused,
        grid=grid,
        in_specs=[
            pl.BlockSpec((_C, _T), lambda i: (0, i)),
            pl.BlockSpec((1, _T), lambda i: (0, i)),
            pl.BlockSpec((_P, _E, _C, _C), lambda i: (0, 0, 0, 0)),
            pl.BlockSpec((_P, _E, _C), lambda i: (0, 0, 0)),
            pl.BlockSpec((_C, _E), lambda i: (0, 0)),
            pl.BlockSpec((_E, _E), lambda i: (0, 0)),
        ],
        out_specs=[
            pl.BlockSpec((_P, _C, _T), lambda i: (0, 0, i)),
            pl.BlockSpec((1, _T), lambda i: (0, i)),
        ],
        out_shape=[
            jax.ShapeDtypeStruct((_P, _C, _N), jnp.float32),
            jax.ShapeDtypeStruct((1, _N), jnp.int32),
        ],
        compiler_params=pltpu.CompilerParams(
            dimension_semantics=("arbitrary",)),
    )(x2, eta2, W, b, Wt, Tmat)
    return out.reshape(_P, _C, _NY, _NX), new_eta.reshape(_NY, _NX)
